# Initial kernel scaffold; baseline (speedup 1.0000x reference)
import jax
import jax.numpy as jnp
from jax.experimental import pallas as pl


def kernel(table, idx):
    raise NotImplementedError("write your pallas kernel here")


# speedup vs baseline: 1.0693x; 1.0693x over previous
import jax
import jax.numpy as jnp
from jax import lax
from jax.experimental import pallas as pl
from jax.experimental.pallas import tpu as pltpu
from jax.experimental.pallas import tpu_sc as plsc

VOCAB = 1000
BATCH = 1024
SEQ = 50

NC = 2
NS = 16
NW = NC * NS

B = BATCH * SEQ
B_PER_W = B // NW
CHUNK = 64
N_CHUNKS = B_PER_W // CHUNK

STAGE_ROWS = 64


def _gather_kernel(table_hbm, idx_hbm, out_hbm, table_sp, idx_v, rows_v, sem):
    cid = lax.axis_index("c")
    sid = lax.axis_index("s")
    wid = sid * NC + cid
    base = wid * B_PER_W

    row0 = sid * STAGE_ROWS
    nrows = jnp.where(sid == NS - 1, VOCAB - (NS - 1) * STAGE_ROWS, STAGE_ROWS)

    @pl.when(sid < NS - 1)
    def _():
        pltpu.sync_copy(
            table_hbm.at[pl.ds(row0, STAGE_ROWS)],
            table_sp.at[pl.ds(row0, STAGE_ROWS)],
        )

    @pl.when(sid == NS - 1)
    def _():
        last0 = (NS - 1) * STAGE_ROWS
        pltpu.sync_copy(
            table_hbm.at[pl.ds(last0, VOCAB - last0)],
            table_sp.at[pl.ds(last0, VOCAB - last0)],
        )

    del nrows
    plsc.subcore_barrier()

    pltpu.sync_copy(idx_hbm.at[pl.ds(base, B_PER_W)], idx_v)

    @pl.loop(0, N_CHUNKS)
    def _(c):
        off = c * CHUNK
        pltpu.async_copy(
            table_sp.at[idx_v.at[pl.ds(off, CHUNK)]], rows_v, sem
        ).wait()
        pltpu.sync_copy(rows_v, out_hbm.at[pl.ds(base + off, CHUNK)])


@jax.jit
def _gather(table, idx_flat):
    mesh = plsc.VectorSubcoreMesh(core_axis_name="c", subcore_axis_name="s")
    k = pl.kernel(
        _gather_kernel,
        out_type=jax.ShapeDtypeStruct((B, VOCAB), jnp.float32),
        mesh=mesh,
        compiler_params=pltpu.CompilerParams(use_tc_tiling_on_sc=False),
        scratch_types=[
            pltpu.VMEM_SHARED((VOCAB, VOCAB), jnp.float32),
            pltpu.VMEM((B_PER_W,), jnp.int32),
            pltpu.VMEM((CHUNK, VOCAB), jnp.float32),
            pltpu.SemaphoreType.DMA,
        ],
    )
    return k(table, idx_flat)


def kernel(table, idx):
    out = _gather(table, idx.reshape(-1))
    return out.reshape(BATCH, SEQ, VOCAB)


# speedup vs baseline: 1.1442x; 1.0701x over previous
import jax
import jax.numpy as jnp
from jax import lax
from jax.experimental import pallas as pl
from jax.experimental.pallas import tpu as pltpu
from jax.experimental.pallas import tpu_sc as plsc

VOCAB = 1000
BATCH = 1024
SEQ = 50

NC = 2
NS = 16
NW = NC * NS

B = BATCH * SEQ
B_PER_W = B // NW
CHUNK = 32
N_CHUNKS = B_PER_W // CHUNK

STAGE_ROWS = 64


def _gather_kernel(
    table_hbm, idx_hbm, out_hbm, table_sp, idx_v, rows0, rows1, sem0, sem1
):
    cid = lax.axis_index("c")
    sid = lax.axis_index("s")
    wid = sid * NC + cid
    base = wid * B_PER_W

    row0 = sid * STAGE_ROWS
    nrows = jnp.where(sid == NS - 1, VOCAB - (NS - 1) * STAGE_ROWS, STAGE_ROWS)

    @pl.when(sid < NS - 1)
    def _():
        pltpu.sync_copy(
            table_hbm.at[pl.ds(row0, STAGE_ROWS)],
            table_sp.at[pl.ds(row0, STAGE_ROWS)],
        )

    @pl.when(sid == NS - 1)
    def _():
        last0 = (NS - 1) * STAGE_ROWS
        pltpu.sync_copy(
            table_hbm.at[pl.ds(last0, VOCAB - last0)],
            table_sp.at[pl.ds(last0, VOCAB - last0)],
        )

    del nrows
    plsc.subcore_barrier()

    pltpu.sync_copy(idx_hbm.at[pl.ds(base, B_PER_W)], idx_v)

    bufs = ((rows0, sem0), (rows1, sem1))

    for b in range(2):
        rows, gs = bufs[b]
        pltpu.make_async_copy(
            table_sp.at[idx_v.at[pl.ds(b * CHUNK, CHUNK)]], rows, gs
        ).start()

    @pl.loop(0, N_CHUNKS // 2)
    def _(p):
        for b in range(2):
            rows, gs = bufs[b]
            c = p * 2 + b
            off = c * CHUNK
            pltpu.make_async_copy(
                table_sp.at[idx_v.at[pl.ds(off, CHUNK)]], rows, gs
            ).wait()
            pltpu.sync_copy(rows, out_hbm.at[pl.ds(base + off, CHUNK)])

            @pl.when(c + 2 < N_CHUNKS)
            def _():
                pltpu.make_async_copy(
                    table_sp.at[idx_v.at[pl.ds(off + 2 * CHUNK, CHUNK)]],
                    rows, gs,
                ).start()


@jax.jit
def _gather(table, idx_flat):
    mesh = plsc.VectorSubcoreMesh(core_axis_name="c", subcore_axis_name="s")
    k = pl.kernel(
        _gather_kernel,
        out_type=jax.ShapeDtypeStruct((B, VOCAB), jnp.float32),
        mesh=mesh,
        compiler_params=pltpu.CompilerParams(use_tc_tiling_on_sc=False),
        scratch_types=[
            pltpu.VMEM_SHARED((VOCAB, VOCAB), jnp.float32),
            pltpu.VMEM((B_PER_W,), jnp.int32),
            pltpu.VMEM((CHUNK, VOCAB), jnp.float32),
            pltpu.VMEM((CHUNK, VOCAB), jnp.float32),
            pltpu.SemaphoreType.DMA,
            pltpu.SemaphoreType.DMA,
        ],
    )
    return k(table, idx_flat)


def kernel(table, idx):
    out = _gather(table, idx.reshape(-1))
    return out.reshape(BATCH, SEQ, VOCAB)


# speedup vs baseline: 1.4175x; 1.2388x over previous
import jax
import jax.numpy as jnp
from jax import lax
from jax.experimental import pallas as pl
from jax.experimental.pallas import tpu as pltpu
from jax.experimental.pallas import tpu_sc as plsc

VOCAB = 1000
VOCAB_PAD = 1024
BATCH = 1024
SEQ = 50

NC = 2
NS = 16
NW = NC * NS

B = BATCH * SEQ
B_PER_W = B // NW
CHUNK = 40
N_CHUNKS = B_PER_W // CHUNK


def _gather_kernel(table_hbm, idx_hbm, out_hbm, idx_v, rows0, rows1, sem0, sem1):
    cid = lax.axis_index("c")
    sid = lax.axis_index("s")
    wid = sid * NC + cid
    base = wid * B_PER_W

    pltpu.sync_copy(idx_hbm.at[pl.ds(base, B_PER_W)], idx_v)

    bufs = ((rows0, sem0), (rows1, sem1))

    for b in range(2):
        rows, gs = bufs[b]
        pltpu.make_async_copy(
            table_hbm.at[idx_v.at[pl.ds(b * CHUNK, CHUNK)]], rows, gs
        ).start()

    @pl.loop(0, N_CHUNKS // 2)
    def _(p):
        for b in range(2):
            rows, gs = bufs[b]
            c = p * 2 + b
            off = c * CHUNK
            pltpu.make_async_copy(
                table_hbm.at[idx_v.at[pl.ds(off, CHUNK)]], rows, gs
            ).wait()
            pltpu.sync_copy(rows, out_hbm.at[pl.ds(base + off, CHUNK)])

            @pl.when(c + 2 < N_CHUNKS)
            def _():
                pltpu.make_async_copy(
                    table_hbm.at[idx_v.at[pl.ds(off + 2 * CHUNK, CHUNK)]],
                    rows, gs,
                ).start()


@jax.jit
def _gather(table_pad, idx_flat):
    mesh = plsc.VectorSubcoreMesh(core_axis_name="c", subcore_axis_name="s")
    k = pl.kernel(
        _gather_kernel,
        out_type=jax.ShapeDtypeStruct((B, VOCAB_PAD), jnp.float32),
        mesh=mesh,
        scratch_types=[
            pltpu.VMEM((B_PER_W,), jnp.int32),
            pltpu.VMEM((CHUNK, VOCAB_PAD), jnp.float32),
            pltpu.VMEM((CHUNK, VOCAB_PAD), jnp.float32),
            pltpu.SemaphoreType.DMA,
            pltpu.SemaphoreType.DMA,
        ],
    )
    return k(table_pad, idx_flat)


def kernel(table, idx):
    table_pad = jnp.pad(table, ((0, 0), (0, VOCAB_PAD - VOCAB)))
    out = _gather(table_pad, idx.reshape(-1))
    return out[:, :VOCAB].reshape(BATCH, SEQ, VOCAB)
